# baseline (speedup 1.0000x reference)
import functools

import jax
import jax.numpy as jnp
from jax import lax
from jax.experimental import pallas as pl
from jax.experimental.pallas import tpu as pltpu
from jax.experimental.pallas import tpu_sc as plsc

N = 10000
D = 128
E = 320000
NC = 2
NS = 16
NW = NC * NS
NBUF = 4
CHUNK = 64
CHUNKS = 160
IBLK = 40
EPT = CHUNK * CHUNKS
EPAD = EPT * NW
NPAD = 10240
ZROWS = NPAD // NS
LN_EPS = 1e-5


def _ln_relu(x, gamma, beta):
    def body(x_ref, g_ref, b_ref, o_ref):
        xv = x_ref[...]
        mu = jnp.mean(xv, axis=-1, keepdims=True)
        xc = xv - mu
        var = jnp.mean(xc * xc, axis=-1, keepdims=True)
        hh = xc * lax.rsqrt(var + LN_EPS) * g_ref[...] + b_ref[...]
        o_ref[...] = jnp.maximum(hh, 0.0)

    blk = 2000
    return pl.pallas_call(
        body,
        grid=(N // blk,),
        in_specs=[
            pl.BlockSpec((blk, D), lambda i: (i, 0)),
            pl.BlockSpec((1, D), lambda i: (0, 0)),
            pl.BlockSpec((1, D), lambda i: (0, 0)),
        ],
        out_specs=pl.BlockSpec((blk, D), lambda i: (i, 0)),
        out_shape=jax.ShapeDtypeStruct((N, D), jnp.float32),
    )(x, gamma[None, :], beta[None, :])


def _sc_segment_sum(h, srcp, dstp):
    mesh = plsc.VectorSubcoreMesh(core_axis_name="c", subcore_axis_name="s",
                                  num_cores=NC)

    @functools.partial(
        pl.kernel,
        out_type=jax.ShapeDtypeStruct((NC, NPAD, D), jnp.float32),
        mesh=mesh,
        scratch_types=[
            pltpu.VMEM((IBLK, CHUNK), jnp.int32),
            pltpu.VMEM((IBLK, CHUNK), jnp.int32),
            [pltpu.VMEM((CHUNK, D), jnp.float32) for _ in range(NBUF)],
            pltpu.VMEM_SHARED((NPAD, D), jnp.float32),
            [pltpu.SemaphoreType.DMA for _ in range(NBUF)],
        ],
    )
    def k(h_hbm, src_hbm, dst_hbm, out_hbm, src_v, dst_v, bufs, agg, sems):
        c = lax.axis_index("c")
        s = lax.axis_index("s")
        wid = c * NS + s

        zero = jnp.zeros((16,), jnp.float32)

        def _zrow(r, carry):
            for kk in range(D // 16):
                bufs[0][r, pl.ds(kk * 16, 16)] = zero
            return carry

        lax.fori_loop(0, CHUNK, _zrow, 0)
        for j in range(ZROWS // CHUNK):
            pltpu.sync_copy(bufs[0], agg.at[pl.ds(s * ZROWS + j * CHUNK, CHUNK)])
        plsc.subcore_barrier()

        for ib in range(CHUNKS // IBLK):
            pltpu.sync_copy(src_hbm.at[wid, pl.ds(ib * IBLK, IBLK)], src_v)
            pltpu.sync_copy(dst_hbm.at[wid, pl.ds(ib * IBLK, IBLK)], dst_v)
            for r in range(NBUF):
                pltpu.async_copy(h_hbm.at[src_v.at[r]], bufs[r], sems[r])

            def _ring(g, carry):
                for r in range(NBUF):
                    i = NBUF * g + r
                    pltpu.make_async_copy(h_hbm.at[src_v.at[i]], bufs[r],
                                          sems[r]).wait()
                    pltpu.sync_copy(bufs[r], agg.at[dst_v.at[i]], add=True)

                    @pl.when(i + NBUF < IBLK)
                    def _():
                        pltpu.async_copy(h_hbm.at[src_v.at[i + NBUF]], bufs[r],
                                         sems[r])
                return carry

            lax.fori_loop(0, IBLK // NBUF, _ring, 0)
        plsc.subcore_barrier()

        pltpu.sync_copy(agg.at[pl.ds(s * ZROWS, ZROWS)],
                        out_hbm.at[c, pl.ds(s * ZROWS, ZROWS)])

    return k(h, srcp, dstp)


def _final(x, parts, gamma, beta, W, b, eps):
    def body(e_ref, x_ref, p_ref, g_ref, be_ref, w_ref, b_ref, o_ref):
        xv = x_ref[...]
        mu = jnp.mean(xv, axis=-1, keepdims=True)
        xc = xv - mu
        var = jnp.mean(xc * xc, axis=-1, keepdims=True)
        hh = jnp.maximum(xc * lax.rsqrt(var + LN_EPS) * g_ref[...] + be_ref[...], 0.0)
        z = (1.0 + e_ref[0]) * hh + p_ref[0] + p_ref[1]
        o = lax.dot_general(z, w_ref[...], (((1,), (1,)), ((), ())),
                            preferred_element_type=jnp.float32)
        o_ref[...] = o + b_ref[...] + xv

    blk = 2000
    return pl.pallas_call(
        body,
        grid=(N // blk,),
        in_specs=[
            pl.BlockSpec(memory_space=pltpu.SMEM),
            pl.BlockSpec((blk, D), lambda i: (i, 0)),
            pl.BlockSpec((NC, blk, D), lambda i: (0, i, 0)),
            pl.BlockSpec((1, D), lambda i: (0, 0)),
            pl.BlockSpec((1, D), lambda i: (0, 0)),
            pl.BlockSpec((D, D), lambda i: (0, 0)),
            pl.BlockSpec((1, D), lambda i: (0, 0)),
        ],
        out_specs=pl.BlockSpec((blk, D), lambda i: (i, 0)),
        out_shape=jax.ShapeDtypeStruct((N, D), jnp.float32),
    )(eps.reshape(1), x, parts, gamma[None, :], beta[None, :], W, b[None, :])


def kernel(x, edge_index, ln_gamma, ln_beta, gine_eps, W, b):
    h = _ln_relu(x, ln_gamma, ln_beta)
    pad = EPAD - E
    srcp = jnp.concatenate([edge_index[0], jnp.zeros((pad,), jnp.int32)])
    dstp = jnp.concatenate([edge_index[1], jnp.full((pad,), N, jnp.int32)])
    srcp = srcp.reshape(NW, CHUNKS, CHUNK)
    dstp = dstp.reshape(NW, CHUNKS, CHUNK)
    parts = _sc_segment_sum(h, srcp, dstp)[:, :N, :]
    return _final(x, parts, ln_gamma, ln_beta, W, b, gine_eps)


# speedup vs baseline: 1.1282x; 1.1282x over previous
import functools

import jax
import jax.numpy as jnp
from jax import lax
from jax.experimental import pallas as pl
from jax.experimental.pallas import tpu as pltpu
from jax.experimental.pallas import tpu_sc as plsc

N = 10000
D = 128
E = 320000
NC = 2
NS = 16
NW = NC * NS
HHALF = 5120
CHUNK = 32
CHUNKS = 320
IBLK = 16
EPT = CHUNK * CHUNKS
EPAD = EPT * NW
NPAD = 10008
DUMMY = N
ZROWS = 632
LAST = N - 15 * ZROWS
HSTG = HHALF // NS
LN_EPS = 1e-5


def _ln_relu(x, gamma, beta):
    def body(x_ref, g_ref, b_ref, o_ref):
        xv = x_ref[...]
        mu = jnp.mean(xv, axis=-1, keepdims=True)
        xc = xv - mu
        var = jnp.mean(xc * xc, axis=-1, keepdims=True)
        hh = xc * lax.rsqrt(var + LN_EPS) * g_ref[...] + b_ref[...]
        o_ref[...] = jnp.maximum(hh, 0.0)

    blk = 2000
    return pl.pallas_call(
        body,
        grid=(N // blk,),
        in_specs=[
            pl.BlockSpec((blk, D), lambda i: (i, 0)),
            pl.BlockSpec((1, D), lambda i: (0, 0)),
            pl.BlockSpec((1, D), lambda i: (0, 0)),
        ],
        out_specs=pl.BlockSpec((blk, D), lambda i: (i, 0)),
        out_shape=jax.ShapeDtypeStruct((N, D), jnp.float32),
    )(x, gamma[None, :], beta[None, :])


def _sc_segment_sum(h, srcp, dstp):
    mesh = plsc.VectorSubcoreMesh(core_axis_name="c", subcore_axis_name="s",
                                  num_cores=NC)

    @functools.partial(
        pl.kernel,
        out_type=jax.ShapeDtypeStruct((NC, NPAD, D), jnp.float32),
        mesh=mesh,
        scratch_types=[
            pltpu.VMEM((IBLK, CHUNK), jnp.int32),
            pltpu.VMEM((IBLK, CHUNK), jnp.int32),
            pltpu.VMEM((CHUNK, D), jnp.float32),
            pltpu.VMEM_SHARED((HHALF, D), jnp.float32),
            pltpu.VMEM_SHARED((NPAD, D), jnp.float32),
        ],
    )
    def k(h_hbm, src_hbm, dst_hbm, out_hbm, src_v, dst_v, buf, hsp, agg):
        c = lax.axis_index("c")
        s = lax.axis_index("s")
        lo = c * HHALF

        for j in range(HSTG // CHUNK):
            row = s * HSTG + j * CHUNK
            full = lo + row + CHUNK <= N

            @pl.when(full)
            def _():
                pltpu.sync_copy(h_hbm.at[pl.ds(lo + row, CHUNK)], buf)
                pltpu.sync_copy(buf, hsp.at[pl.ds(row, CHUNK)])

            @pl.when(jnp.logical_and(jnp.logical_not(full), lo + row < N))
            def _():
                pltpu.sync_copy(h_hbm.at[pl.ds(lo + row, 16)], buf.at[pl.ds(0, 16)])
                pltpu.sync_copy(buf.at[pl.ds(0, 16)], hsp.at[pl.ds(row, 16)])

        zero = jnp.zeros((16,), jnp.float32)

        def _zrow(r, carry):
            for kk in range(D // 16):
                buf[r, pl.ds(kk * 16, 16)] = zero
            return carry

        lax.fori_loop(0, CHUNK, _zrow, 0)

        @pl.when(s < NS - 1)
        def _():
            for j in range(ZROWS // CHUNK):
                pltpu.sync_copy(buf, agg.at[pl.ds(s * ZROWS + j * CHUNK, CHUNK)])
            pltpu.sync_copy(buf.at[pl.ds(0, ZROWS % CHUNK)],
                            agg.at[pl.ds(s * ZROWS + (ZROWS // CHUNK) * CHUNK,
                                         ZROWS % CHUNK)])

        @pl.when(s == NS - 1)
        def _():
            for j in range(LAST // CHUNK):
                pltpu.sync_copy(buf, agg.at[pl.ds(15 * ZROWS + j * CHUNK, CHUNK)])
            pltpu.sync_copy(buf.at[pl.ds(0, LAST % CHUNK)],
                            agg.at[pl.ds(15 * ZROWS + (LAST // CHUNK) * CHUNK,
                                         LAST % CHUNK)])
        plsc.subcore_barrier()

        def _xform(r, carry):
            for kk in range(CHUNK // 16):
                sv = src_v[r, pl.ds(kk * 16, 16)]
                dv = dst_v[r, pl.ds(kk * 16, 16)]
                sl = sv - lo
                m = (sl >= 0) & (sl < HHALF)
                src_v[r, pl.ds(kk * 16, 16)] = jnp.where(m, sl, 0)
                dst_v[r, pl.ds(kk * 16, 16)] = jnp.where(m, dv, DUMMY)
            return carry

        def _chunk(i, carry):
            pltpu.sync_copy(hsp.at[src_v.at[i]], buf)
            pltpu.sync_copy(buf, agg.at[dst_v.at[i]], add=True)
            return carry

        for half in range(2):
            blk = half * NS + s
            for ib in range(CHUNKS // IBLK):
                pltpu.sync_copy(src_hbm.at[blk, pl.ds(ib * IBLK, IBLK)], src_v)
                pltpu.sync_copy(dst_hbm.at[blk, pl.ds(ib * IBLK, IBLK)], dst_v)
                lax.fori_loop(0, IBLK, _xform, 0)
                lax.fori_loop(0, IBLK, _chunk, 0)
        plsc.subcore_barrier()

        @pl.when(s < NS - 1)
        def _():
            pltpu.sync_copy(agg.at[pl.ds(s * ZROWS, ZROWS)],
                            out_hbm.at[c, pl.ds(s * ZROWS, ZROWS)])

        @pl.when(s == NS - 1)
        def _():
            pltpu.sync_copy(agg.at[pl.ds(15 * ZROWS, LAST)],
                            out_hbm.at[c, pl.ds(15 * ZROWS, LAST)])

    return k(h, srcp, dstp)


def _final(x, parts, gamma, beta, W, b, eps):
    def body(e_ref, x_ref, p_ref, g_ref, be_ref, w_ref, b_ref, o_ref):
        xv = x_ref[...]
        mu = jnp.mean(xv, axis=-1, keepdims=True)
        xc = xv - mu
        var = jnp.mean(xc * xc, axis=-1, keepdims=True)
        hh = jnp.maximum(xc * lax.rsqrt(var + LN_EPS) * g_ref[...] + be_ref[...], 0.0)
        z = (1.0 + e_ref[0]) * hh + p_ref[0] + p_ref[1]
        o = lax.dot_general(z, w_ref[...], (((1,), (1,)), ((), ())),
                            preferred_element_type=jnp.float32)
        o_ref[...] = o + b_ref[...] + xv

    blk = 2000
    return pl.pallas_call(
        body,
        grid=(N // blk,),
        in_specs=[
            pl.BlockSpec(memory_space=pltpu.SMEM),
            pl.BlockSpec((blk, D), lambda i: (i, 0)),
            pl.BlockSpec((NC, blk, D), lambda i: (0, i, 0)),
            pl.BlockSpec((1, D), lambda i: (0, 0)),
            pl.BlockSpec((1, D), lambda i: (0, 0)),
            pl.BlockSpec((D, D), lambda i: (0, 0)),
            pl.BlockSpec((1, D), lambda i: (0, 0)),
        ],
        out_specs=pl.BlockSpec((blk, D), lambda i: (i, 0)),
        out_shape=jax.ShapeDtypeStruct((N, D), jnp.float32),
    )(eps.reshape(1), x, parts, gamma[None, :], beta[None, :], W, b[None, :])


def kernel(x, edge_index, ln_gamma, ln_beta, gine_eps, W, b):
    h = _ln_relu(x, ln_gamma, ln_beta)
    pad = EPAD - E
    srcp = jnp.concatenate([edge_index[0], jnp.zeros((pad,), jnp.int32)])
    dstp = jnp.concatenate([edge_index[1], jnp.full((pad,), DUMMY, jnp.int32)])
    srcp = srcp.reshape(NW, CHUNKS, CHUNK)
    dstp = dstp.reshape(NW, CHUNKS, CHUNK)
    parts = _sc_segment_sum(h, srcp, dstp)[:, :N, :]
    return _final(x, parts, ln_gamma, ln_beta, W, b, gine_eps)


# speedup vs baseline: 1.2493x; 1.1073x over previous
import functools

import jax
import jax.numpy as jnp
from jax import lax
from jax.experimental import pallas as pl
from jax.experimental.pallas import tpu as pltpu
from jax.experimental.pallas import tpu_sc as plsc

N = 10000
D = 128
E = 320000
NC = 2
NS = 16
NW = NC * NS
HHALF = 5120
CHUNK = 32
CHUNKS = 320
IBLK = 8
EPT = CHUNK * CHUNKS
EPAD = EPT * NW
NPAD = 10008
DUMMY = N
ZROWS = 632
LAST = N - 15 * ZROWS
HSTG = HHALF // NS
LN_EPS = 1e-5


def _ln_relu(x, gamma, beta):
    def body(x_ref, g_ref, b_ref, o_ref):
        xv = x_ref[...]
        mu = jnp.mean(xv, axis=-1, keepdims=True)
        xc = xv - mu
        var = jnp.mean(xc * xc, axis=-1, keepdims=True)
        hh = xc * lax.rsqrt(var + LN_EPS) * g_ref[...] + b_ref[...]
        o_ref[...] = jnp.maximum(hh, 0.0)

    blk = 2000
    return pl.pallas_call(
        body,
        grid=(N // blk,),
        in_specs=[
            pl.BlockSpec((blk, D), lambda i: (i, 0)),
            pl.BlockSpec((1, D), lambda i: (0, 0)),
            pl.BlockSpec((1, D), lambda i: (0, 0)),
        ],
        out_specs=pl.BlockSpec((blk, D), lambda i: (i, 0)),
        out_shape=jax.ShapeDtypeStruct((N, D), jnp.float32),
    )(x, gamma[None, :], beta[None, :])


def _sc_segment_sum(h, srcp, dstp):
    mesh = plsc.VectorSubcoreMesh(core_axis_name="c", subcore_axis_name="s",
                                  num_cores=NC)

    @functools.partial(
        pl.kernel,
        out_type=jax.ShapeDtypeStruct((NC, NPAD, D), jnp.float32),
        mesh=mesh,
        scratch_types=[
            pltpu.VMEM((2, 128), jnp.int32),
            pltpu.VMEM((IBLK, CHUNK), jnp.int32),
            pltpu.VMEM((CHUNK, D), jnp.float32),
            pltpu.VMEM((CHUNK, D), jnp.float32),
            pltpu.VMEM_SHARED((HHALF, D), jnp.float32),
            pltpu.VMEM_SHARED((NPAD, D), jnp.float32),
            pltpu.SemaphoreType.DMA,
            pltpu.SemaphoreType.DMA,
        ],
    )
    def k(h_hbm, src_hbm, dst_hbm, out_hbm, src_v, dst_v, buf, buf_b, hsp, agg,
          sem_a, sem_b):
        c = lax.axis_index("c")
        s = lax.axis_index("s")
        lo = c * HHALF

        for j in range(HSTG // CHUNK):
            row = s * HSTG + j * CHUNK
            full = lo + row + CHUNK <= N

            @pl.when(full)
            def _():
                pltpu.sync_copy(h_hbm.at[pl.ds(lo + row, CHUNK)], buf)
                pltpu.sync_copy(buf, hsp.at[pl.ds(row, CHUNK)])

            @pl.when(jnp.logical_and(jnp.logical_not(full), lo + row < N))
            def _():
                pltpu.sync_copy(h_hbm.at[pl.ds(lo + row, 16)], buf.at[pl.ds(0, 16)])
                pltpu.sync_copy(buf.at[pl.ds(0, 16)], hsp.at[pl.ds(row, 16)])

        zero = jnp.zeros((16,), jnp.float32)

        def _zrow(r, carry):
            for kk in range(D // 16):
                buf[r, pl.ds(kk * 16, 16)] = zero
            return carry

        lax.fori_loop(0, CHUNK, _zrow, 0)

        @pl.when(s < NS - 1)
        def _():
            for j in range(ZROWS // CHUNK):
                pltpu.sync_copy(buf, agg.at[pl.ds(s * ZROWS + j * CHUNK, CHUNK)])
            pltpu.sync_copy(buf.at[pl.ds(0, ZROWS % CHUNK)],
                            agg.at[pl.ds(s * ZROWS + (ZROWS // CHUNK) * CHUNK,
                                         ZROWS % CHUNK)])

        @pl.when(s == NS - 1)
        def _():
            for j in range(LAST // CHUNK):
                pltpu.sync_copy(buf, agg.at[pl.ds(15 * ZROWS + j * CHUNK, CHUNK)])
            pltpu.sync_copy(buf.at[pl.ds(0, LAST % CHUNK)],
                            agg.at[pl.ds(15 * ZROWS + (LAST // CHUNK) * CHUNK,
                                         LAST % CHUNK)])
        plsc.subcore_barrier()

        def _xform(g, carry):
            sv = src_v[g // 8, pl.ds(16 * (g % 8), 16)]
            dv = dst_v[g // 2, pl.ds(16 * (g % 2), 16)]
            sl = sv - lo
            m = (sl >= 0) & (sl < HHALF)
            src_v[g // 8, pl.ds(16 * (g % 8), 16)] = jnp.where(m, sl, 0)
            dst_v[g // 2, pl.ds(16 * (g % 2), 16)] = jnp.where(m, dv, DUMMY)
            return carry

        def _sidx(i):
            return src_v.at[i // 4, pl.ds(32 * (i % 4), 32)]

        def _step(i, b, sem):
            pltpu.make_async_copy(hsp.at[_sidx(i)], b, sem).wait()
            pltpu.sync_copy(b, agg.at[dst_v.at[i]], add=True)

            @pl.when(i + 2 < IBLK)
            def _():
                pltpu.async_copy(hsp.at[_sidx(i + 2)], b, sem)

        def _pair(g, carry):
            _step(2 * g, buf, sem_a)
            _step(2 * g + 1, buf_b, sem_b)
            return carry

        nblocks = CHUNKS // IBLK

        def _block(t, carry):
            blk = s + NS * (t // nblocks)
            ib = t % nblocks
            pltpu.sync_copy(src_hbm.at[blk, ib], src_v)
            pltpu.sync_copy(dst_hbm.at[blk, ib], dst_v)
            lax.fori_loop(0, 2 * IBLK, _xform, 0)
            pltpu.async_copy(hsp.at[_sidx(0)], buf, sem_a)
            pltpu.async_copy(hsp.at[_sidx(1)], buf_b, sem_b)
            lax.fori_loop(0, IBLK // 2, _pair, 0)
            return carry

        lax.fori_loop(0, 2 * nblocks, _block, 0)
        plsc.subcore_barrier()

        @pl.when(s < NS - 1)
        def _():
            pltpu.sync_copy(agg.at[pl.ds(s * ZROWS, ZROWS)],
                            out_hbm.at[c, pl.ds(s * ZROWS, ZROWS)])

        @pl.when(s == NS - 1)
        def _():
            pltpu.sync_copy(agg.at[pl.ds(15 * ZROWS, LAST)],
                            out_hbm.at[c, pl.ds(15 * ZROWS, LAST)])

    return k(h, srcp, dstp)


def _final(x, parts, gamma, beta, W, b, eps):
    def body(e_ref, x_ref, p_ref, g_ref, be_ref, w_ref, b_ref, o_ref):
        xv = x_ref[...]
        mu = jnp.mean(xv, axis=-1, keepdims=True)
        xc = xv - mu
        var = jnp.mean(xc * xc, axis=-1, keepdims=True)
        hh = jnp.maximum(xc * lax.rsqrt(var + LN_EPS) * g_ref[...] + be_ref[...], 0.0)
        z = (1.0 + e_ref[0]) * hh + p_ref[0] + p_ref[1]
        o = lax.dot_general(z, w_ref[...], (((1,), (1,)), ((), ())),
                            preferred_element_type=jnp.float32)
        o_ref[...] = o + b_ref[...] + xv

    blk = 2000
    return pl.pallas_call(
        body,
        grid=(N // blk,),
        in_specs=[
            pl.BlockSpec(memory_space=pltpu.SMEM),
            pl.BlockSpec((blk, D), lambda i: (i, 0)),
            pl.BlockSpec((NC, blk, D), lambda i: (0, i, 0)),
            pl.BlockSpec((1, D), lambda i: (0, 0)),
            pl.BlockSpec((1, D), lambda i: (0, 0)),
            pl.BlockSpec((D, D), lambda i: (0, 0)),
            pl.BlockSpec((1, D), lambda i: (0, 0)),
        ],
        out_specs=pl.BlockSpec((blk, D), lambda i: (i, 0)),
        out_shape=jax.ShapeDtypeStruct((N, D), jnp.float32),
    )(eps.reshape(1), x, parts, gamma[None, :], beta[None, :], W, b[None, :])


def kernel(x, edge_index, ln_gamma, ln_beta, gine_eps, W, b):
    h = _ln_relu(x, ln_gamma, ln_beta)
    pad = EPAD - E
    srcp = jnp.concatenate([edge_index[0], jnp.zeros((pad,), jnp.int32)])
    dstp = jnp.concatenate([edge_index[1], jnp.full((pad,), DUMMY, jnp.int32)])
    srcp = srcp.reshape(NW, CHUNKS // IBLK, 2, 128)
    dstp = dstp.reshape(NW, CHUNKS // IBLK, IBLK, CHUNK)
    parts = _sc_segment_sum(h, srcp, dstp)[:, :N, :]
    return _final(x, parts, ln_gamma, ln_beta, W, b, gine_eps)


# speedup vs baseline: 1.3235x; 1.0594x over previous
import functools

import jax
import jax.numpy as jnp
from jax import lax
from jax.experimental import pallas as pl
from jax.experimental.pallas import tpu as pltpu
from jax.experimental.pallas import tpu_sc as plsc

N = 10000
D = 128
E = 320000
NC = 2
NS = 16
NW = NC * NS
HHALF = 5120
CHUNK = 32
CHUNKS = 320
IBLK = 8
EPT = CHUNK * CHUNKS
EPAD = EPT * NW
NPAD = 10008
DUMMY = N
ZROWS = 632
LAST = N - 15 * ZROWS
HSTG = HHALF // NS
LN_EPS = 1e-5


def _ln_relu(x, gamma, beta):
    def body(x_ref, g_ref, b_ref, o_ref):
        xv = x_ref[...]
        mu = jnp.mean(xv, axis=-1, keepdims=True)
        xc = xv - mu
        var = jnp.mean(xc * xc, axis=-1, keepdims=True)
        hh = xc * lax.rsqrt(var + LN_EPS) * g_ref[...] + b_ref[...]
        o_ref[...] = jnp.maximum(hh, 0.0)

    blk = 2000
    return pl.pallas_call(
        body,
        grid=(N // blk,),
        in_specs=[
            pl.BlockSpec((blk, D), lambda i: (i, 0)),
            pl.BlockSpec((1, D), lambda i: (0, 0)),
            pl.BlockSpec((1, D), lambda i: (0, 0)),
        ],
        out_specs=pl.BlockSpec((blk, D), lambda i: (i, 0)),
        out_shape=jax.ShapeDtypeStruct((N, D), jnp.float32),
    )(x, gamma[None, :], beta[None, :])


def _sc_segment_sum(h, srcp, dstp):
    mesh = plsc.VectorSubcoreMesh(core_axis_name="c", subcore_axis_name="s",
                                  num_cores=NC)

    @functools.partial(
        pl.kernel,
        out_type=jax.ShapeDtypeStruct((NC, NPAD, D), jnp.float32),
        mesh=mesh,
        scratch_types=[
            pltpu.VMEM((2, 128), jnp.int32),
            pltpu.VMEM((IBLK, CHUNK), jnp.int32),
            pltpu.VMEM((CHUNK, D), jnp.float32),
            pltpu.VMEM((CHUNK, D), jnp.float32),
            pltpu.VMEM_SHARED((HHALF, D), jnp.float32),
            pltpu.VMEM_SHARED((NPAD, D), jnp.float32),
            pltpu.SemaphoreType.DMA,
            pltpu.SemaphoreType.DMA,
            pltpu.SemaphoreType.DMA,
            pltpu.SemaphoreType.DMA,
        ],
    )
    def k(h_hbm, src_hbm, dst_hbm, out_hbm, src_v, dst_v, buf, buf_b, hsp, agg,
          sem_a, sem_b, sem_i, sem_j):
        c = lax.axis_index("c")
        s = lax.axis_index("s")
        lo = c * HHALF

        for j in range(HSTG // CHUNK):
            row = s * HSTG + j * CHUNK
            full = lo + row + CHUNK <= N

            @pl.when(full)
            def _():
                pltpu.sync_copy(h_hbm.at[pl.ds(lo + row, CHUNK)], buf)
                pltpu.sync_copy(buf, hsp.at[pl.ds(row, CHUNK)])

            @pl.when(jnp.logical_and(jnp.logical_not(full), lo + row < N))
            def _():
                pltpu.sync_copy(h_hbm.at[pl.ds(lo + row, 16)], buf.at[pl.ds(0, 16)])
                pltpu.sync_copy(buf.at[pl.ds(0, 16)], hsp.at[pl.ds(row, 16)])

        zero = jnp.zeros((16,), jnp.float32)

        def _zrow(r, carry):
            for kk in range(D // 16):
                buf[r, pl.ds(kk * 16, 16)] = zero
            return carry

        lax.fori_loop(0, CHUNK, _zrow, 0)

        @pl.when(s < NS - 1)
        def _():
            for j in range(ZROWS // CHUNK):
                pltpu.sync_copy(buf, agg.at[pl.ds(s * ZROWS + j * CHUNK, CHUNK)])
            pltpu.sync_copy(buf.at[pl.ds(0, ZROWS % CHUNK)],
                            agg.at[pl.ds(s * ZROWS + (ZROWS // CHUNK) * CHUNK,
                                         ZROWS % CHUNK)])

        @pl.when(s == NS - 1)
        def _():
            for j in range(LAST // CHUNK):
                pltpu.sync_copy(buf, agg.at[pl.ds(15 * ZROWS + j * CHUNK, CHUNK)])
            pltpu.sync_copy(buf.at[pl.ds(0, LAST % CHUNK)],
                            agg.at[pl.ds(15 * ZROWS + (LAST // CHUNK) * CHUNK,
                                         LAST % CHUNK)])
        plsc.subcore_barrier()

        def _xform(g, carry):
            sv = src_v[g // 8, pl.ds(16 * (g % 8), 16)]
            dv = dst_v[g // 2, pl.ds(16 * (g % 2), 16)]
            sl = sv - lo
            m = (sl >= 0) & (sl < HHALF)
            src_v[g // 8, pl.ds(16 * (g % 8), 16)] = jnp.where(m, sl, 0)
            dst_v[g // 2, pl.ds(16 * (g % 2), 16)] = jnp.where(m, dv, DUMMY)
            return carry

        def _sidx(i):
            return src_v.at[i // 4, pl.ds(32 * (i % 4), 32)]

        def _step(i, b, sem):
            pltpu.make_async_copy(hsp.at[_sidx(i)], b, sem).wait()
            pltpu.sync_copy(b, agg.at[dst_v.at[i]], add=True)

            @pl.when(i + 2 < IBLK)
            def _():
                pltpu.async_copy(hsp.at[_sidx(i + 2)], b, sem)

        def _pair(g, carry):
            _step(2 * g, buf, sem_a)
            _step(2 * g + 1, buf_b, sem_b)
            return carry

        nblocks = CHUNKS // IBLK

        def _block(t, carry):
            blk = s + NS * (t // nblocks)
            ib = t % nblocks
            cs = pltpu.async_copy(src_hbm.at[blk, ib], src_v, sem_i)
            cd = pltpu.async_copy(dst_hbm.at[blk, ib], dst_v, sem_j)
            cs.wait()
            cd.wait()
            lax.fori_loop(0, 4, _xform, 0)
            pltpu.async_copy(hsp.at[_sidx(0)], buf, sem_a)
            pltpu.async_copy(hsp.at[_sidx(1)], buf_b, sem_b)
            lax.fori_loop(4, 2 * IBLK, _xform, 0)
            lax.fori_loop(0, IBLK // 2, _pair, 0)
            return carry

        lax.fori_loop(0, 2 * nblocks, _block, 0)
        plsc.subcore_barrier()

        @pl.when(s < NS - 1)
        def _():
            pltpu.sync_copy(agg.at[pl.ds(s * ZROWS, ZROWS)],
                            out_hbm.at[c, pl.ds(s * ZROWS, ZROWS)])

        @pl.when(s == NS - 1)
        def _():
            pltpu.sync_copy(agg.at[pl.ds(15 * ZROWS, LAST)],
                            out_hbm.at[c, pl.ds(15 * ZROWS, LAST)])

    return k(h, srcp, dstp)


def _final(x, parts, gamma, beta, W, b, eps):
    def body(e_ref, x_ref, p_ref, g_ref, be_ref, w_ref, b_ref, o_ref):
        xv = x_ref[...]
        mu = jnp.mean(xv, axis=-1, keepdims=True)
        xc = xv - mu
        var = jnp.mean(xc * xc, axis=-1, keepdims=True)
        hh = jnp.maximum(xc * lax.rsqrt(var + LN_EPS) * g_ref[...] + be_ref[...], 0.0)
        z = (1.0 + e_ref[0]) * hh + p_ref[0] + p_ref[1]
        o = lax.dot_general(z, w_ref[...], (((1,), (1,)), ((), ())),
                            preferred_element_type=jnp.float32)
        o_ref[...] = o + b_ref[...] + xv

    blk = 2000
    return pl.pallas_call(
        body,
        grid=(N // blk,),
        in_specs=[
            pl.BlockSpec(memory_space=pltpu.SMEM),
            pl.BlockSpec((blk, D), lambda i: (i, 0)),
            pl.BlockSpec((NC, blk, D), lambda i: (0, i, 0)),
            pl.BlockSpec((1, D), lambda i: (0, 0)),
            pl.BlockSpec((1, D), lambda i: (0, 0)),
            pl.BlockSpec((D, D), lambda i: (0, 0)),
            pl.BlockSpec((1, D), lambda i: (0, 0)),
        ],
        out_specs=pl.BlockSpec((blk, D), lambda i: (i, 0)),
        out_shape=jax.ShapeDtypeStruct((N, D), jnp.float32),
    )(eps.reshape(1), x, parts, gamma[None, :], beta[None, :], W, b[None, :])


def kernel(x, edge_index, ln_gamma, ln_beta, gine_eps, W, b):
    h = _ln_relu(x, ln_gamma, ln_beta)
    pad = EPAD - E
    srcp = jnp.concatenate([edge_index[0], jnp.zeros((pad,), jnp.int32)])
    dstp = jnp.concatenate([edge_index[1], jnp.full((pad,), DUMMY, jnp.int32)])
    srcp = srcp.reshape(NW, CHUNKS // IBLK, 2, 128)
    dstp = dstp.reshape(NW, CHUNKS // IBLK, IBLK, CHUNK)
    parts = _sc_segment_sum(h, srcp, dstp)[:, :N, :]
    return _final(x, parts, ln_gamma, ln_beta, W, b, gine_eps)
